# baseline (speedup 1.0000x reference)
import functools

import jax
import jax.numpy as jnp
from jax import lax
from jax.experimental import pallas as pl
from jax.experimental.pallas import tpu as pltpu
import jax.experimental.pallas.tpu_sc as plsc

NW = 32
CHUNK = 128
GROUP = 5
NGROUP = 10
NCHUNK = GROUP * NGROUP
P = 16
ALPHA = 4
D = 64


def _sc_gather(ids3, u_lin, v_table):
    T = NW * NCHUNK * CHUNK
    GC = GROUP * CHUNK
    vocab = u_lin.shape[0] // P
    mesh = plsc.VectorSubcoreMesh(
        core_axis_name="c", subcore_axis_name="s", num_cores=2, num_subcores=16)

    @functools.partial(
        pl.kernel,
        out_type=(jax.ShapeDtypeStruct((T * P,), jnp.float32),
                  jax.ShapeDtypeStruct((T, P), jnp.float32),
                  jax.ShapeDtypeStruct((T, P * ALPHA), jnp.float32)),
        mesh=mesh,
        compiler_params=pltpu.CompilerParams(
            use_tc_tiling_on_sc=False, needs_layout_passes=False),
        scratch_types=[
            pltpu.VMEM((NCHUNK, CHUNK), jnp.int32),
            pltpu.VMEM((GC * P,), jnp.int32),
            pltpu.VMEM((GC * P,), jnp.float32),
            pltpu.VMEM((GC, P), jnp.float32),
            pltpu.VMEM((GC, P * ALPHA), jnp.bfloat16),
            pltpu.VMEM((GC, P * ALPHA), jnp.float32),
            pltpu.SemaphoreType.DMA,
            pltpu.SemaphoreType.DMA,
            pltpu.SemaphoreType.DMA,
            pltpu.SemaphoreType.DMA,
            pltpu.SemaphoreType.DMA,
        ],
    )
    def gather_kernel(ids_hbm, u_hbm, v_hbm, u_out, f_out, y_out,
                      ids_v, idxu_v, u_v, f_v, v_v, y_v,
                      su, sv, swu, swf, swy):
        wid = lax.axis_index("s") * 2 + lax.axis_index("c")
        base = wid * (NCHUNK * CHUNK)
        lanes = lax.iota(jnp.int32, 16)
        idx_gate0 = lanes // 2
        idx_gate1 = idx_gate0 + 8
        idx_e0 = lanes * 2
        idx_o0 = idx_e0 + 1
        idx_e1 = idx_e0 + 32
        idx_o1 = idx_e1 + 1
        pltpu.sync_copy(ids_hbm.at[wid], ids_v)

        def group_body(g, carry):
            handles = []
            for k in range(GROUP):
                idx = ids_v.at[g * GROUP + k]
                handles.append(pltpu.async_copy(
                    v_hbm.at[idx], v_v.at[pl.ds(k * CHUNK, CHUNK)], sv))

            def idx_body(i, c):
                tb = plsc.load_gather(
                    ids_v, [jnp.full((16,), g * GROUP + i // CHUNK, jnp.int32),
                            jnp.full((16,), i % CHUNK, jnp.int32)])
                idxu_v[pl.ds(i * P, P)] = tb + lanes * vocab
                return c

            lax.fori_loop(0, GC, idx_body, 0)
            hu = pltpu.async_copy(u_hbm.at[idxu_v], u_v, su)
            for h in handles:
                h.wait()
            hu.wait()
            wu = pltpu.async_copy(
                u_v, u_out.at[pl.ds((base + g * GC) * P, GC * P)], swu)

            def token_body(i, c):
                u = u_v[pl.ds(i * P, P)]
                f = jnp.where(u > 0.0, jnp.float32(1.0), jnp.float32(0.0))
                f_v[i] = f
                row = jnp.full((16,), i, dtype=jnp.int32)
                g0 = plsc.load_gather(f_v, [row, idx_gate0])
                g1 = plsc.load_gather(f_v, [row, idx_gate1])
                c0 = plsc.bitcast(v_v[i, pl.ds(0, 32)], jnp.int32)
                c1 = plsc.bitcast(v_v[i, pl.ds(32, 32)], jnp.int32)
                e0 = plsc.bitcast(c0 << 16, jnp.float32)
                o0 = plsc.bitcast(c0 & jnp.int32(-65536), jnp.float32)
                e1 = plsc.bitcast(c1 << 16, jnp.float32)
                o1 = plsc.bitcast(c1 & jnp.int32(-65536), jnp.float32)
                plsc.store_scatter(y_v, [row, idx_e0], e0 * g0)
                plsc.store_scatter(y_v, [row, idx_o0], o0 * g0)
                plsc.store_scatter(y_v, [row, idx_e1], e1 * g1)
                plsc.store_scatter(y_v, [row, idx_o1], o1 * g1)
                return c

            lax.fori_loop(0, GC, token_body, 0)
            off = base + g * GC
            wf = pltpu.async_copy(f_v, f_out.at[pl.ds(off, GC)], swf)
            wy = pltpu.async_copy(y_v, y_out.at[pl.ds(off, GC)], swy)
            wu.wait()
            wf.wait()
            wy.wait()
            return carry

        lax.fori_loop(0, NGROUP, group_body, 0)

    return gather_kernel(ids3, u_lin, v_table)


def _tc_project(yf, wd, bb):
    R = yf.shape[0]
    RB = 1024

    def body(y_ref, wd_ref, b_ref, x_ref):
        x_ref[...] = jnp.dot(y_ref[...], wd_ref[...],
                             preferred_element_type=jnp.float32,
                             precision=lax.Precision.HIGHEST) + b_ref[...]

    return pl.pallas_call(
        body,
        grid=(R // RB,),
        in_specs=[pl.BlockSpec((RB, 128), lambda i: (i, 0)),
                  pl.BlockSpec((128, 128), lambda i: (0, 0)),
                  pl.BlockSpec((1, 128), lambda i: (0, 0))],
        out_specs=pl.BlockSpec((RB, 128), lambda i: (i, 0)),
        out_shape=jax.ShapeDtypeStruct((R, 128), jnp.float32),
    )(yf, wd, bb)


def kernel(input_ids, U_table, V_table, W, b):
    B, L = input_ids.shape
    T = B * L
    assert T == NW * NCHUNK * CHUNK
    ids3 = input_ids.reshape(NW, NCHUNK, CHUNK).astype(jnp.int32)
    u_lin = U_table.T.reshape(-1)
    u1d, f_flat, y_flat = _sc_gather(
        ids3, u_lin, V_table.reshape(V_table.shape[0], P * ALPHA))
    u_flat = u1d.reshape(T, P)
    yf = y_flat.reshape(T * D // 128, 128)
    wt = W.T
    z = jnp.zeros((D, D), dtype=jnp.float32)
    wd = jnp.concatenate(
        [jnp.concatenate([wt, z], axis=1), jnp.concatenate([z, wt], axis=1)],
        axis=0)
    bb = jnp.concatenate([b, b]).reshape(1, 2 * D)
    x2 = _tc_project(yf, wd, bb)
    return (x2.reshape(B, L, D), u_flat.reshape(B, L, P),
            f_flat.reshape(B, L, P), y_flat.reshape(B, L, P, ALPHA))


# speedup vs baseline: 1.5436x; 1.5436x over previous
import functools

import jax
import jax.numpy as jnp
from jax import lax
from jax.experimental import pallas as pl
from jax.experimental.pallas import tpu as pltpu
import jax.experimental.pallas.tpu_sc as plsc

NW = 32
CHUNK = 128
GROUP = 5
NGROUP = 10
NCHUNK = GROUP * NGROUP
P = 16
ALPHA = 4
D = 64


def _sc_gather(ids3, u_table, v_table):
    T = NW * NCHUNK * CHUNK
    GC = GROUP * CHUNK
    mesh = plsc.VectorSubcoreMesh(
        core_axis_name="c", subcore_axis_name="s", num_cores=2, num_subcores=16)

    @functools.partial(
        pl.kernel,
        out_type=(jax.ShapeDtypeStruct((T, P), jnp.float32),
                  jax.ShapeDtypeStruct((T, P), jnp.float32),
                  jax.ShapeDtypeStruct((T, P * ALPHA), jnp.float32)),
        mesh=mesh,
        compiler_params=pltpu.CompilerParams(
            use_tc_tiling_on_sc=False, needs_layout_passes=False),
        scratch_types=[
            pltpu.VMEM((NCHUNK, CHUNK), jnp.int32),
            pltpu.VMEM((2, GC, P), jnp.float32),
            pltpu.VMEM((GC, P), jnp.float32),
            pltpu.VMEM((2, GC, P * ALPHA), jnp.bfloat16),
            pltpu.VMEM((GC, P * ALPHA), jnp.float32),
            pltpu.SemaphoreType.DMA,
            pltpu.SemaphoreType.DMA,
            pltpu.SemaphoreType.DMA,
            pltpu.SemaphoreType.DMA,
            pltpu.SemaphoreType.DMA,
        ],
    )
    def gather_kernel(ids_hbm, u_hbm, v_hbm, u_out, f_out, y_out,
                      ids_v, u_v, f_v, v_v, y_v, su, sv, swu, swf, swy):
        wid = lax.axis_index("s") * 2 + lax.axis_index("c")
        base = wid * (NCHUNK * CHUNK)
        lanes = lax.iota(jnp.int32, 16)
        idx_gate0 = lanes // 2
        idx_gate1 = idx_gate0 + 8
        idx_e0 = lanes * 2
        idx_o0 = idx_e0 + 1
        idx_e1 = idx_e0 + 32
        idx_o1 = idx_e1 + 1
        pltpu.sync_copy(ids_hbm.at[wid], ids_v)

        def issue_gathers(g, buf):
            for k in range(GROUP):
                idx = ids_v.at[g * GROUP + k]
                pltpu.async_copy(
                    u_hbm.at[idx], u_v.at[buf].at[pl.ds(k * CHUNK, CHUNK)], su)
                pltpu.async_copy(
                    v_hbm.at[idx], v_v.at[buf].at[pl.ds(k * CHUNK, CHUNK)], sv)

        issue_gathers(0, 0)

        def group_body(g, carry):
            buf = lax.rem(g, 2)
            for k in range(GROUP):
                idx = ids_v.at[g * GROUP + k]
                pltpu.make_async_copy(
                    u_hbm.at[idx], u_v.at[buf].at[pl.ds(k * CHUNK, CHUNK)],
                    su).wait()
                pltpu.make_async_copy(
                    v_hbm.at[idx], v_v.at[buf].at[pl.ds(k * CHUNK, CHUNK)],
                    sv).wait()
            wu = pltpu.async_copy(
                u_v.at[buf], u_out.at[pl.ds(base + g * GC, GC)], swu)

            @pl.when(g + 1 < NGROUP)
            def _():
                issue_gathers(g + 1, 1 - buf)

            u_b = u_v.at[buf]
            v_b = v_v.at[buf]

            def token_body(i, c):
                u = u_b[i]
                f = jnp.where(u > 0.0, jnp.float32(1.0), jnp.float32(0.0))
                f_v[i] = f
                row = jnp.full((16,), i, dtype=jnp.int32)
                g0 = plsc.load_gather(f_v, [row, idx_gate0])
                g1 = plsc.load_gather(f_v, [row, idx_gate1])
                c0 = plsc.bitcast(v_b[i, pl.ds(0, 32)], jnp.int32)
                c1 = plsc.bitcast(v_b[i, pl.ds(32, 32)], jnp.int32)
                e0 = plsc.bitcast(c0 << 16, jnp.float32)
                o0 = plsc.bitcast(c0 & jnp.int32(-65536), jnp.float32)
                e1 = plsc.bitcast(c1 << 16, jnp.float32)
                o1 = plsc.bitcast(c1 & jnp.int32(-65536), jnp.float32)
                plsc.store_scatter(y_v, [row, idx_e0], e0 * g0)
                plsc.store_scatter(y_v, [row, idx_o0], o0 * g0)
                plsc.store_scatter(y_v, [row, idx_e1], e1 * g1)
                plsc.store_scatter(y_v, [row, idx_o1], o1 * g1)
                return c

            lax.fori_loop(0, GC, token_body, 0)
            off = base + g * GC
            wf = pltpu.async_copy(f_v, f_out.at[pl.ds(off, GC)], swf)
            wy = pltpu.async_copy(y_v, y_out.at[pl.ds(off, GC)], swy)
            wu.wait()
            wf.wait()
            wy.wait()
            return carry

        lax.fori_loop(0, NGROUP, group_body, 0)

    return gather_kernel(ids3, u_table, v_table)


def _tc_project(yf, wd, bb):
    R = yf.shape[0]
    RB = 1024

    def body(y_ref, wd_ref, b_ref, x_ref):
        x_ref[...] = jnp.dot(y_ref[...], wd_ref[...],
                             preferred_element_type=jnp.float32,
                             precision=lax.Precision.HIGHEST) + b_ref[...]

    return pl.pallas_call(
        body,
        grid=(R // RB,),
        in_specs=[pl.BlockSpec((RB, 128), lambda i: (i, 0)),
                  pl.BlockSpec((128, 128), lambda i: (0, 0)),
                  pl.BlockSpec((1, 128), lambda i: (0, 0))],
        out_specs=pl.BlockSpec((RB, 128), lambda i: (i, 0)),
        out_shape=jax.ShapeDtypeStruct((R, 128), jnp.float32),
    )(yf, wd, bb)


def kernel(input_ids, U_table, V_table, W, b):
    B, L = input_ids.shape
    T = B * L
    assert T == NW * NCHUNK * CHUNK
    ids3 = input_ids.reshape(NW, NCHUNK, CHUNK).astype(jnp.int32)
    u_flat, f_flat, y_flat = _sc_gather(
        ids3, U_table, V_table.reshape(V_table.shape[0], P * ALPHA))
    yf = y_flat.reshape(T * D // 128, 128)
    wt = W.T
    z = jnp.zeros((D, D), dtype=jnp.float32)
    wd = jnp.concatenate(
        [jnp.concatenate([wt, z], axis=1), jnp.concatenate([z, wt], axis=1)],
        axis=0)
    bb = jnp.concatenate([b, b]).reshape(1, 2 * D)
    x2 = _tc_project(yf, wd, bb)
    return (x2.reshape(B, L, D), u_flat.reshape(B, L, P),
            f_flat.reshape(B, L, P), y_flat.reshape(B, L, P, ALPHA))


# speedup vs baseline: 1.5448x; 1.0008x over previous
import functools

import jax
import jax.numpy as jnp
from jax import lax
from jax.experimental import pallas as pl
from jax.experimental.pallas import tpu as pltpu
import jax.experimental.pallas.tpu_sc as plsc

NW = 32
CHUNK = 128
GROUP = 5
NGROUP = 10
NCHUNK = GROUP * NGROUP
P = 16
ALPHA = 4
D = 64


def _sc_gather(ids3, u_table, v_table):
    T = NW * NCHUNK * CHUNK
    GC = GROUP * CHUNK
    mesh = plsc.VectorSubcoreMesh(
        core_axis_name="c", subcore_axis_name="s", num_cores=2, num_subcores=16)

    @functools.partial(
        pl.kernel,
        out_type=(jax.ShapeDtypeStruct((T, P), jnp.float32),
                  jax.ShapeDtypeStruct((T, P), jnp.float32),
                  jax.ShapeDtypeStruct((T, P * ALPHA), jnp.float32)),
        mesh=mesh,
        compiler_params=pltpu.CompilerParams(
            use_tc_tiling_on_sc=False, needs_layout_passes=False),
        scratch_types=[
            pltpu.VMEM((NCHUNK, CHUNK), jnp.int32),
            pltpu.VMEM((2, GC, P), jnp.float32),
            pltpu.VMEM((GC, P), jnp.float32),
            pltpu.VMEM((2, GC, P * ALPHA), jnp.bfloat16),
            pltpu.VMEM((GC, P * ALPHA), jnp.float32),
            pltpu.SemaphoreType.DMA,
            pltpu.SemaphoreType.DMA,
            pltpu.SemaphoreType.DMA,
            pltpu.SemaphoreType.DMA,
            pltpu.SemaphoreType.DMA,
        ],
    )
    def gather_kernel(ids_hbm, u_hbm, v_hbm, u_out, f_out, y_out,
                      ids_v, u_v, f_v, v_v, y_v, su, sv, swu, swf, swy):
        wid = lax.axis_index("s") * 2 + lax.axis_index("c")
        base = wid * (NCHUNK * CHUNK)
        lanes = lax.iota(jnp.int32, 16)
        idx_gate0 = lanes // 2
        idx_gate1 = idx_gate0 + 8
        idx_e0 = lanes * 2
        idx_o0 = idx_e0 + 1
        idx_e1 = idx_e0 + 32
        idx_o1 = idx_e1 + 1
        pltpu.sync_copy(ids_hbm.at[wid], ids_v)

        def issue_gathers(g, buf):
            for k in range(GROUP):
                idx = ids_v.at[g * GROUP + k]
                pltpu.async_copy(
                    u_hbm.at[idx], u_v.at[buf].at[pl.ds(k * CHUNK, CHUNK)], su)
                pltpu.async_copy(
                    v_hbm.at[idx], v_v.at[buf].at[pl.ds(k * CHUNK, CHUNK)], sv)

        issue_gathers(0, 0)

        def group_body(g, carry):
            buf = lax.rem(g, 2)
            for k in range(GROUP):
                idx = ids_v.at[g * GROUP + k]
                pltpu.make_async_copy(
                    u_hbm.at[idx], u_v.at[buf].at[pl.ds(k * CHUNK, CHUNK)],
                    su).wait()
                pltpu.make_async_copy(
                    v_hbm.at[idx], v_v.at[buf].at[pl.ds(k * CHUNK, CHUNK)],
                    sv).wait()
            wu = pltpu.async_copy(
                u_v.at[buf], u_out.at[pl.ds(base + g * GC, GC)], swu)

            @pl.when(g + 1 < NGROUP)
            def _():
                issue_gathers(g + 1, 1 - buf)

            u_b = u_v.at[buf]
            v_b = v_v.at[buf]

            def one_token(i):
                u = u_b[i]
                f = jnp.where(u > 0.0, jnp.float32(1.0), jnp.float32(0.0))
                f_v[i] = f
                row = jnp.full((16,), i, dtype=jnp.int32)
                g0 = plsc.load_gather(f_v, [row, idx_gate0])
                g1 = plsc.load_gather(f_v, [row, idx_gate1])
                c0 = plsc.bitcast(v_b[i, pl.ds(0, 32)], jnp.int32)
                c1 = plsc.bitcast(v_b[i, pl.ds(32, 32)], jnp.int32)
                e0 = plsc.bitcast(c0 << 16, jnp.float32)
                o0 = plsc.bitcast(c0 & jnp.int32(-65536), jnp.float32)
                e1 = plsc.bitcast(c1 << 16, jnp.float32)
                o1 = plsc.bitcast(c1 & jnp.int32(-65536), jnp.float32)
                plsc.store_scatter(y_v, [row, idx_e0], e0 * g0)
                plsc.store_scatter(y_v, [row, idx_o0], o0 * g0)
                plsc.store_scatter(y_v, [row, idx_e1], e1 * g1)
                plsc.store_scatter(y_v, [row, idx_o1], o1 * g1)

            def token_body(j, c):
                for s in range(4):
                    one_token(4 * j + s)
                return c

            lax.fori_loop(0, GC // 4, token_body, 0)
            off = base + g * GC
            wf = pltpu.async_copy(f_v, f_out.at[pl.ds(off, GC)], swf)
            wy = pltpu.async_copy(y_v, y_out.at[pl.ds(off, GC)], swy)
            wu.wait()
            wf.wait()
            wy.wait()
            return carry

        lax.fori_loop(0, NGROUP, group_body, 0)

    return gather_kernel(ids3, u_table, v_table)


def _tc_project(yf, wd, bb):
    R = yf.shape[0]
    RB = 1024

    def body(y_ref, wd_ref, b_ref, x_ref):
        x_ref[...] = jnp.dot(y_ref[...], wd_ref[...],
                             preferred_element_type=jnp.float32,
                             precision=lax.Precision.HIGHEST) + b_ref[...]

    return pl.pallas_call(
        body,
        grid=(R // RB,),
        in_specs=[pl.BlockSpec((RB, 128), lambda i: (i, 0)),
                  pl.BlockSpec((128, 128), lambda i: (0, 0)),
                  pl.BlockSpec((1, 128), lambda i: (0, 0))],
        out_specs=pl.BlockSpec((RB, 128), lambda i: (i, 0)),
        out_shape=jax.ShapeDtypeStruct((R, 128), jnp.float32),
    )(yf, wd, bb)


def kernel(input_ids, U_table, V_table, W, b):
    B, L = input_ids.shape
    T = B * L
    assert T == NW * NCHUNK * CHUNK
    ids3 = input_ids.reshape(NW, NCHUNK, CHUNK).astype(jnp.int32)
    u_flat, f_flat, y_flat = _sc_gather(
        ids3, U_table, V_table.reshape(V_table.shape[0], P * ALPHA))
    yf = y_flat.reshape(T * D // 128, 128)
    wt = W.T
    z = jnp.zeros((D, D), dtype=jnp.float32)
    wd = jnp.concatenate(
        [jnp.concatenate([wt, z], axis=1), jnp.concatenate([z, wt], axis=1)],
        axis=0)
    bb = jnp.concatenate([b, b]).reshape(1, 2 * D)
    x2 = _tc_project(yf, wd, bb)
    return (x2.reshape(B, L, D), u_flat.reshape(B, L, P),
            f_flat.reshape(B, L, P), y_flat.reshape(B, L, P, ALPHA))
